# Initial kernel scaffold; baseline (speedup 1.0000x reference)
import jax
import jax.numpy as jnp
from jax.experimental import pallas as pl


def kernel(phoneme_ids, tone_ids, language_ids, bert_feats, phoneme_table, tone_table, language_table, W_bert):
    raise NotImplementedError("write your pallas kernel here")


# speedup vs baseline: 1.1994x; 1.1994x over previous
import functools

import jax
import jax.numpy as jnp
from jax import lax
from jax.experimental import pallas as pl
from jax.experimental.pallas import tpu as pltpu
from jax.experimental.pallas import tpu_sc as plsc


def _sc_dims():
    try:
        info = plsc.get_sparse_core_info()
        return info.num_cores, info.num_subcores
    except Exception:
        return 2, 16


_LANES = 16


def _sc_embed_sum(ptab, ttab, ltab, pids, tids, lids, *, chunk):
    n_tok, h = pids.shape[0], ptab.shape[1]
    nc, ns = _sc_dims()
    nw = nc * ns
    assert n_tok % (nw * chunk) == 0
    per_w = n_tok // nw
    n_chunks = per_w // chunk
    mesh = plsc.VectorSubcoreMesh(core_axis_name="c", subcore_axis_name="s")

    @functools.partial(
        pl.kernel,
        mesh=mesh,
        out_type=jax.ShapeDtypeStruct((n_tok, h), jnp.float32),
        scratch_types=[
            pltpu.VMEM((per_w,), jnp.int32),
            pltpu.VMEM((per_w,), jnp.int32),
            pltpu.VMEM((per_w,), jnp.int32),
            pltpu.VMEM((chunk, h), jnp.float32),
            pltpu.VMEM((chunk, h), jnp.float32),
            pltpu.VMEM((chunk, h), jnp.float32),
            pltpu.SemaphoreType.DMA,
            pltpu.SemaphoreType.DMA,
            pltpu.SemaphoreType.DMA,
        ],
    )
    def k(ptab_hbm, ttab_hbm, ltab_hbm, pids_hbm, tids_hbm, lids_hbm,
          out_hbm, pidx_v, tidx_v, lidx_v, rows_p, rows_t, rows_l,
          sem_p, sem_t, sem_l):
        wid = lax.axis_index("s") * nc + lax.axis_index("c")
        base = wid * per_w
        pltpu.sync_copy(pids_hbm.at[pl.ds(base, per_w)], pidx_v)
        pltpu.sync_copy(tids_hbm.at[pl.ds(base, per_w)], tidx_v)
        pltpu.sync_copy(lids_hbm.at[pl.ds(base, per_w)], lidx_v)

        def chunk_body(c, _):
            off = c * chunk
            cp = pltpu.async_copy(
                ptab_hbm.at[pidx_v.at[pl.ds(off, chunk)]], rows_p, sem_p)
            ct = pltpu.async_copy(
                ttab_hbm.at[tidx_v.at[pl.ds(off, chunk)]], rows_t, sem_t)
            cl = pltpu.async_copy(
                ltab_hbm.at[lidx_v.at[pl.ds(off, chunk)]], rows_l, sem_l)
            cp.wait()
            ct.wait()
            cl.wait()

            def row_body(r, _):
                for j in range(h // _LANES):
                    s = pl.ds(j * _LANES, _LANES)
                    rows_p[r, s] = rows_p[r, s] + rows_t[r, s] + rows_l[r, s]
                return 0

            lax.fori_loop(0, chunk, row_body, 0)
            pltpu.sync_copy(rows_p, out_hbm.at[pl.ds(base + off, chunk)])
            return 0

        lax.fori_loop(0, n_chunks, chunk_body, 0)

    return k(ptab, ttab, ltab, pids, tids, lids)


def _tc_proj_body(w_ref, bert_ref, emb_ref, out_ref):
    acc = jnp.dot(w_ref[...], bert_ref[0], preferred_element_type=jnp.float32)
    out_ref[0] = acc + emb_ref[0].T


def _tc_proj_add(w, bert, emb, *, t_blk):
    b, d, t = bert.shape
    h = w.shape[0]
    grid = (b, t // t_blk)
    return pl.pallas_call(
        _tc_proj_body,
        grid=grid,
        in_specs=[
            pl.BlockSpec((h, d), lambda i, j: (0, 0)),
            pl.BlockSpec((1, d, t_blk), lambda i, j: (i, 0, j)),
            pl.BlockSpec((1, t_blk, h), lambda i, j: (i, j, 0)),
        ],
        out_specs=pl.BlockSpec((1, h, t_blk), lambda i, j: (i, 0, j)),
        out_shape=jax.ShapeDtypeStruct((b, h, t), jnp.float32),
    )(w, bert, emb)


def kernel(phoneme_ids, tone_ids, language_ids, bert_feats,
           phoneme_table, tone_table, language_table, W_bert):
    b, t = phoneme_ids.shape
    h = phoneme_table.shape[1]
    n = b * t
    pids = phoneme_ids.reshape(n).astype(jnp.int32)
    tids = tone_ids.reshape(n).astype(jnp.int32)
    lids = language_ids.reshape(n).astype(jnp.int32)

    emb = _sc_embed_sum(phoneme_table, tone_table, language_table,
                        pids, tids, lids, chunk=64)
    emb = emb.reshape(b, t, h)
    return _tc_proj_add(W_bert, bert_feats, emb, t_blk=512)


# speedup vs baseline: 1.4070x; 1.1731x over previous
import functools

import jax
import jax.numpy as jnp
from jax import lax
from jax.experimental import pallas as pl
from jax.experimental.pallas import tpu as pltpu
from jax.experimental.pallas import tpu_sc as plsc


def _sc_dims():
    try:
        info = plsc.get_sparse_core_info()
        return info.num_cores, info.num_subcores
    except Exception:
        return 2, 16


def _sc_gather_rows(tab, ids, *, chunk, nbuf=3):
    n_tok, h = ids.shape[0], tab.shape[1]
    nc, ns = _sc_dims()
    nw = nc * ns
    assert n_tok % (nw * chunk) == 0
    per_w = n_tok // nw
    n_chunks = per_w // chunk
    mesh = plsc.VectorSubcoreMesh(core_axis_name="c", subcore_axis_name="s")

    @functools.partial(
        pl.kernel,
        mesh=mesh,
        out_type=jax.ShapeDtypeStruct((n_tok, h), tab.dtype),
        scratch_types=(
            [pltpu.VMEM((per_w,), jnp.int32)]
            + [pltpu.VMEM((chunk, h), tab.dtype) for _ in range(nbuf)]
            + [pltpu.SemaphoreType.DMA for _ in range(2 * nbuf)]
        ),
    )
    def k(tab_hbm, ids_hbm, out_hbm, idx_v, *rest):
        bufs, sems = rest[:nbuf], rest[nbuf:]
        gsem, wsem = sems[:nbuf], sems[nbuf:]
        wid = lax.axis_index("s") * nc + lax.axis_index("c")
        base = wid * per_w
        pltpu.sync_copy(ids_hbm.at[pl.ds(base, per_w)], idx_v)

        def gstart(c):
            return pltpu.async_copy(
                tab_hbm.at[idx_v.at[pl.ds(c * chunk, chunk)]],
                bufs[c % nbuf], gsem[c % nbuf])

        g = [None] * n_chunks
        w = [None] * n_chunks
        for c in range(min(nbuf - 1, n_chunks)):
            g[c] = gstart(c)
        for c in range(n_chunks):
            g[c].wait()
            w[c] = pltpu.async_copy(
                bufs[c % nbuf], out_hbm.at[pl.ds(base + c * chunk, chunk)],
                wsem[c % nbuf])
            nxt = c + nbuf - 1
            if nxt < n_chunks:
                if nxt >= nbuf:
                    w[nxt - nbuf].wait()
                g[nxt] = gstart(nxt)
        for c in range(max(0, n_chunks - nbuf), n_chunks):
            w[c].wait()

    return k(tab, ids)


def _tc_body(t_blk, w_ref, mt_ref, bert_ref, tid_ref, lid_ref, emb_ref, out_ref):
    bert_bf = bert_ref[0].astype(jnp.bfloat16)
    acc = jnp.dot(w_ref[...], bert_bf, preferred_element_type=jnp.float32)
    iota = lax.broadcasted_iota(jnp.int32, (128, t_blk), 0)
    oh = ((iota == tid_ref[0]) | (iota == lid_ref[0] + 16))
    acc = acc + jnp.dot(mt_ref[...], oh.astype(jnp.bfloat16),
                        preferred_element_type=jnp.float32)
    out_ref[0] = acc + emb_ref[0].T.astype(jnp.float32)


def _tc_proj_add(w_bf, mt_bf, bert, tids3, lids3, emb, *, t_blk):
    b, d, t = bert.shape
    h = w_bf.shape[0]
    grid = (b, t // t_blk)
    return pl.pallas_call(
        functools.partial(_tc_body, t_blk),
        grid=grid,
        in_specs=[
            pl.BlockSpec((h, d), lambda i, j: (0, 0)),
            pl.BlockSpec((h, 128), lambda i, j: (0, 0)),
            pl.BlockSpec((1, d, t_blk), lambda i, j: (i, 0, j)),
            pl.BlockSpec((1, 1, t_blk), lambda i, j: (i, 0, j)),
            pl.BlockSpec((1, 1, t_blk), lambda i, j: (i, 0, j)),
            pl.BlockSpec((1, t_blk, h), lambda i, j: (i, j, 0)),
        ],
        out_specs=pl.BlockSpec((1, h, t_blk), lambda i, j: (i, 0, j)),
        out_shape=jax.ShapeDtypeStruct((b, h, t), jnp.float32),
    )(w_bf, mt_bf, bert, tids3, lids3, emb)


def kernel(phoneme_ids, tone_ids, language_ids, bert_feats,
           phoneme_table, tone_table, language_table, W_bert):
    b, t = phoneme_ids.shape
    h = phoneme_table.shape[1]
    n = b * t
    pids = phoneme_ids.reshape(n).astype(jnp.int32)
    tids3 = tone_ids.reshape(b, 1, t).astype(jnp.int32)
    lids3 = language_ids.reshape(b, 1, t).astype(jnp.int32)

    n_tone, n_lang = tone_table.shape[0], language_table.shape[0]
    mt = jnp.zeros((128, h), jnp.float32)
    mt = mt.at[:n_tone].set(tone_table).at[16:16 + n_lang].set(language_table)
    mt_bf = mt.T.astype(jnp.bfloat16)
    w_bf = W_bert.astype(jnp.bfloat16)
    ptab_bf = phoneme_table.astype(jnp.bfloat16)

    ptab_i32 = lax.bitcast_convert_type(
        ptab_bf.reshape(ptab_bf.shape[0], h // 2, 2), jnp.int32)
    emb_i32 = _sc_gather_rows(ptab_i32, pids, chunk=128)
    emb = lax.bitcast_convert_type(emb_i32, jnp.bfloat16).reshape(b, t, h)
    return _tc_proj_add(w_bf, mt_bf, bert_feats, tids3, lids3, emb, t_blk=512)


# speedup vs baseline: 3.0297x; 2.1533x over previous
import functools

import jax
import jax.numpy as jnp
from jax import lax
from jax.experimental import pallas as pl
from jax.experimental.pallas import tpu as pltpu
from jax.experimental.pallas import tpu_sc as plsc


def _sc_dims():
    try:
        info = plsc.get_sparse_core_info()
        return info.num_cores, info.num_subcores
    except Exception:
        return 2, 16


def _sc_gather_rows(tab, ids, *, chunk, nbuf=3):
    n_tok, h = ids.shape[0], tab.shape[1]
    nc, ns = _sc_dims()
    nw = nc * ns
    assert n_tok % (nw * chunk) == 0
    per_w = n_tok // nw
    n_chunks = per_w // chunk
    mesh = plsc.VectorSubcoreMesh(core_axis_name="c", subcore_axis_name="s")

    @functools.partial(
        pl.kernel,
        mesh=mesh,
        out_type=jax.ShapeDtypeStruct((n_tok, h), tab.dtype),
        scratch_types=(
            [pltpu.VMEM((per_w,), jnp.int32)]
            + [pltpu.VMEM((chunk, h), tab.dtype) for _ in range(nbuf)]
            + [pltpu.SemaphoreType.DMA for _ in range(2 * nbuf)]
        ),
    )
    def k(tab_hbm, ids_hbm, out_hbm, idx_v, *rest):
        bufs, sems = rest[:nbuf], rest[nbuf:]
        gsem, wsem = sems[:nbuf], sems[nbuf:]
        wid = lax.axis_index("s") * nc + lax.axis_index("c")
        base = wid * per_w
        pltpu.sync_copy(ids_hbm.at[pl.ds(base, per_w)], idx_v)

        def gstart(c):
            return pltpu.async_copy(
                tab_hbm.at[idx_v.at[pl.ds(c * chunk, chunk)]],
                bufs[c % nbuf], gsem[c % nbuf])

        g = [None] * n_chunks
        w = [None] * n_chunks
        for c in range(min(nbuf - 1, n_chunks)):
            g[c] = gstart(c)
        for c in range(n_chunks):
            g[c].wait()
            w[c] = pltpu.async_copy(
                bufs[c % nbuf], out_hbm.at[pl.ds(base + c * chunk, chunk)],
                wsem[c % nbuf])
            nxt = c + nbuf - 1
            if nxt < n_chunks:
                if nxt >= nbuf:
                    w[nxt - nbuf].wait()
                g[nxt] = gstart(nxt)
        for c in range(max(0, n_chunks - nbuf), n_chunks):
            w[c].wait()

    return k(tab, ids)


def _tc_body(t_blk, w_ref, mt_ref, bert_ref, tid_ref, lid_ref, emb_ref, out_ref):
    bert_bf = bert_ref[0].astype(jnp.bfloat16)
    acc = jnp.dot(w_ref[...], bert_bf, preferred_element_type=jnp.float32)
    iota = lax.broadcasted_iota(jnp.int32, (128, t_blk), 0)
    oh = ((iota == tid_ref[0]) | (iota == lid_ref[0] + 16))
    acc = acc + jnp.dot(mt_ref[...], oh.astype(jnp.bfloat16),
                        preferred_element_type=jnp.float32)
    out_ref[0] = acc + emb_ref[...].T


def _tc_proj_add(w_bf, mt_bf, bert, tids3, lids3, emb, *, t_blk):
    b, d, t = bert.shape
    h = w_bf.shape[0]
    grid = (b, t // t_blk)
    return pl.pallas_call(
        functools.partial(_tc_body, t_blk),
        grid=grid,
        in_specs=[
            pl.BlockSpec((h, d), lambda i, j: (0, 0)),
            pl.BlockSpec((h, 128), lambda i, j: (0, 0)),
            pl.BlockSpec((1, d, t_blk), lambda i, j: (i, 0, j)),
            pl.BlockSpec((1, 1, t_blk), lambda i, j: (i, 0, j)),
            pl.BlockSpec((1, 1, t_blk), lambda i, j: (i, 0, j)),
            pl.BlockSpec((t_blk, h), lambda i, j, _nt=t // t_blk: (i * _nt + j, 0)),
        ],
        out_specs=pl.BlockSpec((1, h, t_blk), lambda i, j: (i, 0, j)),
        out_shape=jax.ShapeDtypeStruct((b, h, t), jnp.float32),
    )(w_bf, mt_bf, bert, tids3, lids3, emb)


def kernel(phoneme_ids, tone_ids, language_ids, bert_feats,
           phoneme_table, tone_table, language_table, W_bert):
    b, t = phoneme_ids.shape
    h = phoneme_table.shape[1]
    n = b * t
    pids = phoneme_ids.reshape(n).astype(jnp.int32)
    tids3 = tone_ids.reshape(b, 1, t).astype(jnp.int32)
    lids3 = language_ids.reshape(b, 1, t).astype(jnp.int32)

    n_tone, n_lang = tone_table.shape[0], language_table.shape[0]
    mt = jnp.zeros((128, h), jnp.float32)
    mt = mt.at[:n_tone].set(tone_table).at[16:16 + n_lang].set(language_table)
    mt_bf = mt.T.astype(jnp.bfloat16)
    w_bf = W_bert.astype(jnp.bfloat16)

    emb = _sc_gather_rows(phoneme_table, pids, chunk=64)
    return _tc_proj_add(w_bf, mt_bf, bert_feats, tids3, lids3, emb, t_blk=512)


# speedup vs baseline: 3.3522x; 1.1064x over previous
import functools

import jax
import jax.numpy as jnp
from jax import lax
from jax.experimental import pallas as pl
from jax.experimental.pallas import tpu as pltpu
from jax.experimental.pallas import tpu_sc as plsc


def _sc_dims():
    try:
        info = plsc.get_sparse_core_info()
        return info.num_cores, info.num_subcores
    except Exception:
        return 2, 16


def _sc_gather_rows(tab, ids, *, chunk, nbuf=3):
    n_tok, h = ids.shape[0], tab.shape[1]
    nc, ns = _sc_dims()
    nw = nc * ns
    assert n_tok % (nw * chunk) == 0
    per_w = n_tok // nw
    n_chunks = per_w // chunk
    mesh = plsc.VectorSubcoreMesh(core_axis_name="c", subcore_axis_name="s")

    @functools.partial(
        pl.kernel,
        mesh=mesh,
        out_type=jax.ShapeDtypeStruct((n_tok, h), tab.dtype),
        scratch_types=(
            [pltpu.VMEM((per_w,), jnp.int32)]
            + [pltpu.VMEM((chunk, h), tab.dtype) for _ in range(nbuf)]
            + [pltpu.SemaphoreType.DMA for _ in range(2 * nbuf)]
        ),
    )
    def k(tab_hbm, ids_hbm, out_hbm, idx_v, *rest):
        bufs, sems = rest[:nbuf], rest[nbuf:]
        gsem, wsem = sems[:nbuf], sems[nbuf:]
        wid = lax.axis_index("s") * nc + lax.axis_index("c")
        base = wid * per_w
        pltpu.sync_copy(ids_hbm.at[pl.ds(base, per_w)], idx_v)

        def gstart(c):
            return pltpu.async_copy(
                tab_hbm.at[idx_v.at[pl.ds(c * chunk, chunk)]],
                bufs[c % nbuf], gsem[c % nbuf])

        g = [None] * n_chunks
        w = [None] * n_chunks
        for c in range(min(nbuf - 1, n_chunks)):
            g[c] = gstart(c)
        for c in range(n_chunks):
            g[c].wait()
            w[c] = pltpu.async_copy(
                bufs[c % nbuf], out_hbm.at[pl.ds(base + c * chunk, chunk)],
                wsem[c % nbuf])
            nxt = c + nbuf - 1
            if nxt < n_chunks:
                if nxt >= nbuf:
                    w[nxt - nbuf].wait()
                g[nxt] = gstart(nxt)
        for c in range(max(0, n_chunks - nbuf), n_chunks):
            w[c].wait()

    return k(tab, ids)


def _tc_body(t_blk, w_ref, mt_ref, bert_ref, tid_ref, lid_ref, emb_ref, out_ref):
    bert_bf = bert_ref[0].astype(jnp.bfloat16)
    acc = jnp.dot(w_ref[...], bert_bf, preferred_element_type=jnp.float32)
    iota = lax.broadcasted_iota(jnp.int32, (128, t_blk), 0)
    oh = ((iota == tid_ref[0]) | (iota == lid_ref[0] + 16))
    acc = acc + jnp.dot(mt_ref[...], oh.astype(jnp.bfloat16),
                        preferred_element_type=jnp.float32)
    out_ref[0] = acc + emb_ref[...].T


def _tc_proj_add(w_bf, mt_bf, bert, tids3, lids3, emb, *, t_blk):
    b, d, t = bert.shape
    h = w_bf.shape[0]
    grid = (b, t // t_blk)
    return pl.pallas_call(
        functools.partial(_tc_body, t_blk),
        grid=grid,
        in_specs=[
            pl.BlockSpec((h, d), lambda i, j: (0, 0)),
            pl.BlockSpec((h, 128), lambda i, j: (0, 0)),
            pl.BlockSpec((1, d, t_blk), lambda i, j: (i, 0, j)),
            pl.BlockSpec((1, 1, t_blk), lambda i, j: (i, 0, j)),
            pl.BlockSpec((1, 1, t_blk), lambda i, j: (i, 0, j)),
            pl.BlockSpec((t_blk, h), lambda i, j, _nt=t // t_blk: (i * _nt + j, 0)),
        ],
        out_specs=pl.BlockSpec((1, h, t_blk), lambda i, j: (i, 0, j)),
        out_shape=jax.ShapeDtypeStruct((b, h, t), jnp.float32),
        compiler_params=pltpu.CompilerParams(
            dimension_semantics=("parallel", "parallel")),
    )(w_bf, mt_bf, bert, tids3, lids3, emb)


def kernel(phoneme_ids, tone_ids, language_ids, bert_feats,
           phoneme_table, tone_table, language_table, W_bert):
    b, t = phoneme_ids.shape
    h = phoneme_table.shape[1]
    n = b * t
    pids = phoneme_ids.reshape(n).astype(jnp.int32)
    tids3 = tone_ids.reshape(b, 1, t).astype(jnp.int32)
    lids3 = language_ids.reshape(b, 1, t).astype(jnp.int32)

    n_tone, n_lang = tone_table.shape[0], language_table.shape[0]
    mt = jnp.zeros((128, h), jnp.float32)
    mt = mt.at[:n_tone].set(tone_table).at[16:16 + n_lang].set(language_table)
    mt_bf = mt.T.astype(jnp.bfloat16)
    w_bf = W_bert.astype(jnp.bfloat16)

    emb = _sc_gather_rows(phoneme_table, pids, chunk=64)
    return _tc_proj_add(w_bf, mt_bf, bert_feats, tids3, lids3, emb, t_blk=1024)
